# baseline (speedup 1.0000x reference)
import functools

import jax
import jax.numpy as jnp
from jax import lax
from jax.experimental import pallas as pl
from jax.experimental.pallas import tpu as pltpu
from jax.experimental.pallas import tpu_sc as plsc

N_NODES = 10000
N_PAD = 10240
K = 128
NC = 2
NS = 16
NW = NC * NS
ROWS_PT = N_PAD // NS
F32 = jnp.float32


def _ceil_to(a, m):
    return (a + m - 1) // m * m


def _make_deg_kernel(e_pad):
    ept = e_pad // NW
    chunks = ept // K
    seg = N_PAD // NS

    @functools.partial(
        pl.kernel,
        out_type=jax.ShapeDtypeStruct((NC, N_PAD), F32),
        mesh=plsc.VectorSubcoreMesh(core_axis_name="c", subcore_axis_name="s"),
        scratch_types=[
            pltpu.VMEM((K,), jnp.int32),
            pltpu.VMEM((N_PAD,), F32),
            pltpu.VMEM((seg,), F32),
            pltpu.VMEM((NS, seg), F32),
            pltpu.VMEM_SHARED((NS, N_PAD), F32),
        ],
        compiler_params=pltpu.CompilerParams(needs_layout_passes=False),
    )
    def deg_kernel(dst_hbm, out_hbm, idx_v, deg_local, out_local, slab_v, sh):
        cid = lax.axis_index("c")
        sid = lax.axis_index("s")
        wid = sid * NC + cid
        base = wid * ept

        zeros16 = jnp.zeros((16,), F32)
        ones16 = jnp.ones((16,), F32)

        def zbody(i, _):
            deg_local[pl.ds(i * 16, 16)] = zeros16
            return 0
        lax.fori_loop(0, N_PAD // 16, zbody, 0)

        def chunk(c, _):
            pltpu.sync_copy(dst_hbm.at[pl.ds(base + c * K, K)], idx_v)
            for j in range(K // 16):
                idx = idx_v[pl.ds(j * 16, 16)]
                plsc.addupdate_scatter(deg_local, [idx], ones16)
            return 0
        lax.fori_loop(0, chunks, chunk, 0)

        pltpu.sync_copy(deg_local, sh.at[sid])
        plsc.subcore_barrier()

        for j in range(NS):
            pltpu.sync_copy(sh.at[j, pl.ds(sid * seg, seg)], slab_v.at[j])

        def rbody(k, _):
            acc = jnp.zeros((16,), F32)
            for j in range(NS):
                acc = acc + slab_v[j, pl.ds(k * 16, 16)]
            out_local[pl.ds(k * 16, 16)] = acc
            return 0
        lax.fori_loop(0, seg // 16, rbody, 0)

        pltpu.sync_copy(out_local, out_hbm.at[cid, pl.ds(sid * seg, seg)])

    return deg_kernel


def _make_agg_kernel(e_pad, C):
    KA = 64
    ept = e_pad // NW
    chunks = ept // KA
    NB = 4
    assert chunks % (2 * NB) == 0
    giter = chunks // (2 * NB)
    WB = ROWS_PT // KA

    @functools.partial(
        pl.kernel,
        out_type=jax.ShapeDtypeStruct((NC, N_PAD, C), F32),
        mesh=plsc.VectorSubcoreMesh(core_axis_name="c", subcore_axis_name="s"),
        scratch_types=[
            pltpu.VMEM((2 * NB, KA), jnp.int32),
            pltpu.VMEM((2 * NB, KA), jnp.int32),
            pltpu.VMEM((NB, KA, C), F32),
            pltpu.VMEM_SHARED((N_PAD, C), F32),
            pltpu.SemaphoreType.DMA((2 * NB,)),
            pltpu.SemaphoreType.DMA((NB,)),
            pltpu.SemaphoreType.DMA((NB,)),
        ],
    )
    def agg_kernel(y_hbm, src_hbm, dst_hbm, out_hbm, srcx_v, dstx_v, rows_v,
                   acc_sh, si, sg, ss):
        cid = lax.axis_index("c")
        sid = lax.axis_index("s")
        wid = sid * NC + cid
        base_e = wid * ept
        tb = sid * ROWS_PT

        zeros16 = jnp.zeros((16,), F32)

        def zbody(i, _):
            for j in range(C // 16):
                rows_v[0, i, pl.ds(j * 16, 16)] = zeros16
            return 0
        lax.fori_loop(0, KA, zbody, 0)

        for k in range(WB):
            pltpu.sync_copy(rows_v.at[0], acc_sh.at[pl.ds(tb + k * KA, KA)])
        plsc.subcore_barrier()

        def idx_cp(c, b):
            return [pltpu.make_async_copy(
                src_hbm.at[pl.ds(base_e + c * KA, KA)], srcx_v.at[b],
                si.at[b]), pltpu.make_async_copy(
                dst_hbm.at[pl.ds(base_e + c * KA, KA)], dstx_v.at[b],
                si.at[b])]

        def gat_cp(b, r):
            return pltpu.make_async_copy(
                y_hbm.at[srcx_v.at[b]], rows_v.at[r], sg.at[r])

        def sca_cp(b, r):
            return pltpu.make_async_copy(
                rows_v.at[r], acc_sh.at[dstx_v.at[b]], ss.at[r])

        def body(g, _):
            base = g * 2 * NB
            @pl.when(g > 0)
            def _():
                for b in range(NB):
                    sca_cp(NB + b, b).wait()
            for b in range(2 * NB):
                for cp in idx_cp(base + b, b):
                    cp.start()
            for b in range(NB):
                for cp in idx_cp(base + b, b):
                    cp.wait()
                gat_cp(b, b).start()
            for b in range(NB):
                gat_cp(b, b).wait()
                sca_cp(b, b).start(add=True)
            for b in range(NB):
                sca_cp(b, b).wait()
                for cp in idx_cp(base + NB + b, NB + b):
                    cp.wait()
                gat_cp(NB + b, b).start()
            for b in range(NB):
                gat_cp(NB + b, b).wait()
                sca_cp(NB + b, b).start(add=True)
            return 0
        lax.fori_loop(0, giter, body, 0)
        for b in range(NB):
            sca_cp(NB + b, b).wait()

        plsc.subcore_barrier()

        def st_cp(k, r):
            return pltpu.make_async_copy(
                rows_v.at[r], out_hbm.at[cid, pl.ds(tb + k * KA, KA)],
                sg.at[r])

        for k in range(WB):
            r = k % 2
            if k >= 2:
                st_cp(k - 2, r).wait()
            pltpu.sync_copy(acc_sh.at[pl.ds(tb + k * KA, KA)], rows_v.at[r])
            st_cp(k, r).start()
        for k in (WB - 2, WB - 1):
            st_cp(k, k % 2).wait()

    return agg_kernel


def _dis_body(dp_ref, dis_ref):
    dp = dp_ref[...]
    deg = dp[0:1, :] + dp[1:2, :] + 1.0
    dis_ref[...] = lax.rsqrt(deg)


def _y_body(x_ref, w_ref, d_ref, y_ref):
    xw = jnp.dot(x_ref[...], w_ref[...], preferred_element_type=F32)
    y_ref[0:N_NODES, :] = xw * d_ref[0:N_NODES, :]
    y_ref[N_NODES:N_PAD, :] = jnp.zeros((N_PAD - N_NODES, y_ref.shape[1]), F32)


def _comb_body(a0_ref, a1_ref, y_ref, d_ref, b_ref, g_ref, be_ref, w_ref,
               o_ref):
    d = d_ref[0:N_NODES, :]
    t = (a0_ref[0:N_NODES, :] + a1_ref[0:N_NODES, :] + y_ref[0:N_NODES, :]) \
        * d + b_ref[...]
    mean = jnp.mean(t, axis=0, keepdims=True)
    tc = t - mean
    var = jnp.mean(tc * tc, axis=0, keepdims=True)
    h = g_ref[...] * tc * lax.rsqrt(var + 1e-5) + be_ref[...]
    h = jnp.maximum(h, 0.0)
    o_ref[0:N_NODES, :] = jnp.dot(h, w_ref[...], preferred_element_type=F32) \
        * d
    o_ref[N_NODES:N_PAD, :] = jnp.zeros((N_PAD - N_NODES, o_ref.shape[1]), F32)


def _final_body(a0_ref, a1_ref, y_ref, d_ref, b_ref, o_ref):
    C = o_ref.shape[1]
    o_ref[...] = (a0_ref[0:N_NODES, 0:C] + a1_ref[0:N_NODES, 0:C]
                  + y_ref[0:N_NODES, 0:C]) * d_ref[0:N_NODES, :] + b_ref[...]


def _tc(body, out_shape, *args):
    return pl.pallas_call(body, out_shape=out_shape)(*args)


def kernel(x, edge_index, W1, b1, g1, be1, W2, b2, g2, be2, W3, b3):
    N = x.shape[0]
    E = edge_index.shape[1]
    assert N == N_NODES

    ei = edge_index.astype(jnp.int32)
    e_pad = _ceil_to(E, NW * K * 10)
    pad = e_pad - E
    padv = jnp.full((pad,), N, jnp.int32)
    src = jnp.concatenate([ei[0], padv])
    dst = jnp.concatenate([ei[1], padv])

    deg_kernel = _make_deg_kernel(e_pad)
    agg128 = _make_agg_kernel(e_pad, 128)
    agg_lat = _make_agg_kernel(e_pad, 128)

    deg_p = deg_kernel(dst)
    dis = _tc(_dis_body, jax.ShapeDtypeStruct((1, N_PAD), F32), deg_p)
    disc = dis.reshape(N_PAD, 1)

    def layer_mid(h_in_y, a_p, b, g, be, Wn):
        C_out = Wn.shape[1]
        return _tc(_comb_body, jax.ShapeDtypeStruct((N_PAD, C_out), F32),
                   a_p[0], a_p[1], h_in_y, disc, b.reshape(1, -1),
                   g.reshape(1, -1), be.reshape(1, -1), Wn)

    lat = W3.shape[1]
    W3p = jnp.zeros((W3.shape[0], 128), F32).at[:, :lat].set(W3)

    y1 = _tc(_y_body, jax.ShapeDtypeStruct((N_PAD, 128), F32), x, W1, disc)
    a1 = agg128(y1, src, dst)
    y2 = layer_mid(y1, a1, b1, g1, be1, W2)
    a2 = agg128(y2, src, dst)
    y3 = layer_mid(y2, a2, b2, g2, be2, W3p)
    a3 = agg_lat(y3, src, dst)
    out = _tc(_final_body, jax.ShapeDtypeStruct((N_NODES, lat), F32),
              a3[0], a3[1], y3, disc, b3.reshape(1, -1))
    return out


# speedup vs baseline: 1.0865x; 1.0865x over previous
import functools

import jax
import jax.numpy as jnp
from jax import lax
from jax.experimental import pallas as pl
from jax.experimental.pallas import tpu as pltpu
from jax.experimental.pallas import tpu_sc as plsc

N_NODES = 10000
N_PAD = 10240
K = 128
NC = 2
NS = 16
NW = NC * NS
ROWS_PT = N_PAD // NS
F32 = jnp.float32


def _ceil_to(a, m):
    return (a + m - 1) // m * m


def _make_deg_kernel(e_pad):
    ept = e_pad // NW
    chunks = ept // K
    seg = N_PAD // NS

    @functools.partial(
        pl.kernel,
        out_type=jax.ShapeDtypeStruct((NC, N_PAD), F32),
        mesh=plsc.VectorSubcoreMesh(core_axis_name="c", subcore_axis_name="s"),
        scratch_types=[
            pltpu.VMEM((K,), jnp.int32),
            pltpu.VMEM((N_PAD,), F32),
            pltpu.VMEM((seg,), F32),
            pltpu.VMEM((NS, seg), F32),
            pltpu.VMEM_SHARED((NS, N_PAD), F32),
        ],
        compiler_params=pltpu.CompilerParams(needs_layout_passes=False),
    )
    def deg_kernel(dst_hbm, out_hbm, idx_v, deg_local, out_local, slab_v, sh):
        cid = lax.axis_index("c")
        sid = lax.axis_index("s")
        wid = sid * NC + cid
        base = wid * ept

        zeros16 = jnp.zeros((16,), F32)
        ones16 = jnp.ones((16,), F32)

        def zbody(i, _):
            deg_local[pl.ds(i * 16, 16)] = zeros16
            return 0
        lax.fori_loop(0, N_PAD // 16, zbody, 0)

        def chunk(c, _):
            pltpu.sync_copy(dst_hbm.at[pl.ds(base + c * K, K)], idx_v)
            for j in range(K // 16):
                idx = idx_v[pl.ds(j * 16, 16)]
                plsc.addupdate_scatter(deg_local, [idx], ones16)
            return 0
        lax.fori_loop(0, chunks, chunk, 0)

        pltpu.sync_copy(deg_local, sh.at[sid])
        plsc.subcore_barrier()

        for j in range(NS):
            pltpu.sync_copy(sh.at[j, pl.ds(sid * seg, seg)], slab_v.at[j])

        def rbody(k, _):
            acc = jnp.zeros((16,), F32)
            for j in range(NS):
                acc = acc + slab_v[j, pl.ds(k * 16, 16)]
            out_local[pl.ds(k * 16, 16)] = acc
            return 0
        lax.fori_loop(0, seg // 16, rbody, 0)

        pltpu.sync_copy(out_local, out_hbm.at[cid, pl.ds(sid * seg, seg)])

    return deg_kernel


def _make_agg_kernel(e_pad, C, share0=0.75):
    KA = 64
    NB = 4
    T = e_pad // KA
    T0 = _ceil_to(int(T * share0), NS * 2 * NB)
    T1 = T - T0
    assert T1 >= 0 and T1 % (NS * 2 * NB) == 0
    cpt = (T0 // NS, T1 // NS)
    giters = (cpt[0] // (2 * NB), cpt[1] // (2 * NB))
    WB = ROWS_PT // KA

    @functools.partial(
        pl.kernel,
        out_type=jax.ShapeDtypeStruct((NC, N_PAD, C), F32),
        mesh=plsc.VectorSubcoreMesh(core_axis_name="c", subcore_axis_name="s"),
        scratch_types=[
            pltpu.VMEM((2 * NB, KA), jnp.int32),
            pltpu.VMEM((2 * NB, KA), jnp.int32),
            pltpu.VMEM((NB, KA, C), F32),
            pltpu.VMEM_SHARED((N_PAD, C), F32),
            pltpu.SemaphoreType.DMA((2 * NB,)),
            pltpu.SemaphoreType.DMA((NB,)),
            pltpu.SemaphoreType.DMA((NB,)),
        ],
    )
    def agg_kernel(y_hbm, src_hbm, dst_hbm, out_hbm, srcx_v, dstx_v, rows_v,
                   acc_sh, si, sg, ss):
        cid = lax.axis_index("c")
        sid = lax.axis_index("s")
        base_c = jnp.where(cid == 0, sid * cpt[0], T0 + sid * cpt[1])
        base_e = base_c * KA
        giter = jnp.where(cid == 0, giters[0], giters[1])
        tb = sid * ROWS_PT

        zeros16 = jnp.zeros((16,), F32)

        def zbody(i, _):
            for j in range(C // 16):
                rows_v[0, i, pl.ds(j * 16, 16)] = zeros16
            return 0
        lax.fori_loop(0, KA, zbody, 0)

        for k in range(WB):
            pltpu.sync_copy(rows_v.at[0], acc_sh.at[pl.ds(tb + k * KA, KA)])
        plsc.subcore_barrier()

        def idx_cp(c, b):
            return [pltpu.make_async_copy(
                src_hbm.at[pl.ds(base_e + c * KA, KA)], srcx_v.at[b],
                si.at[b]), pltpu.make_async_copy(
                dst_hbm.at[pl.ds(base_e + c * KA, KA)], dstx_v.at[b],
                si.at[b])]

        def gat_cp(b, r):
            return pltpu.make_async_copy(
                y_hbm.at[srcx_v.at[b]], rows_v.at[r], sg.at[r])

        def sca_cp(b, r):
            return pltpu.make_async_copy(
                rows_v.at[r], acc_sh.at[dstx_v.at[b]], ss.at[r])

        def body(g, _):
            base = g * 2 * NB
            @pl.when(g > 0)
            def _():
                for b in range(NB):
                    sca_cp(NB + b, b).wait()
            for b in range(2 * NB):
                for cp in idx_cp(base + b, b):
                    cp.start()
            for b in range(NB):
                for cp in idx_cp(base + b, b):
                    cp.wait()
                gat_cp(b, b).start()
            for b in range(NB):
                gat_cp(b, b).wait()
                sca_cp(b, b).start(add=True)
            for b in range(NB):
                sca_cp(b, b).wait()
                for cp in idx_cp(base + NB + b, NB + b):
                    cp.wait()
                gat_cp(NB + b, b).start()
            for b in range(NB):
                gat_cp(NB + b, b).wait()
                sca_cp(NB + b, b).start(add=True)
            return 0
        lax.fori_loop(0, giter, body, 0)
        for b in range(NB):
            sca_cp(NB + b, b).wait()

        plsc.subcore_barrier()

        def st_cp(k, r):
            return pltpu.make_async_copy(
                rows_v.at[r], out_hbm.at[cid, pl.ds(tb + k * KA, KA)],
                sg.at[r])

        for k in range(WB):
            r = k % 2
            if k >= 2:
                st_cp(k - 2, r).wait()
            pltpu.sync_copy(acc_sh.at[pl.ds(tb + k * KA, KA)], rows_v.at[r])
            st_cp(k, r).start()
        for k in (WB - 2, WB - 1):
            st_cp(k, k % 2).wait()

    return agg_kernel


def _dis_body(dp_ref, dis_ref):
    dp = dp_ref[...]
    deg = dp[0:1, :] + dp[1:2, :] + 1.0
    dis_ref[...] = lax.rsqrt(deg)


def _y_body(x_ref, w_ref, d_ref, y_ref):
    xw = jnp.dot(x_ref[...], w_ref[...], preferred_element_type=F32)
    y_ref[0:N_NODES, :] = xw * d_ref[0:N_NODES, :]
    y_ref[N_NODES:N_PAD, :] = jnp.zeros((N_PAD - N_NODES, y_ref.shape[1]), F32)


def _comb_body(a0_ref, a1_ref, y_ref, d_ref, b_ref, g_ref, be_ref, w_ref,
               o_ref):
    d = d_ref[0:N_NODES, :]
    t = (a0_ref[0:N_NODES, :] + a1_ref[0:N_NODES, :] + y_ref[0:N_NODES, :]) \
        * d + b_ref[...]
    mean = jnp.mean(t, axis=0, keepdims=True)
    tc = t - mean
    var = jnp.mean(tc * tc, axis=0, keepdims=True)
    h = g_ref[...] * tc * lax.rsqrt(var + 1e-5) + be_ref[...]
    h = jnp.maximum(h, 0.0)
    o_ref[0:N_NODES, :] = jnp.dot(h, w_ref[...], preferred_element_type=F32) \
        * d
    o_ref[N_NODES:N_PAD, :] = jnp.zeros((N_PAD - N_NODES, o_ref.shape[1]), F32)


def _final_body(a0_ref, a1_ref, y_ref, d_ref, b_ref, o_ref):
    C = o_ref.shape[1]
    o_ref[...] = (a0_ref[0:N_NODES, 0:C] + a1_ref[0:N_NODES, 0:C]
                  + y_ref[0:N_NODES, 0:C]) * d_ref[0:N_NODES, :] + b_ref[...]


def _tc(body, out_shape, *args):
    return pl.pallas_call(body, out_shape=out_shape)(*args)


def kernel(x, edge_index, W1, b1, g1, be1, W2, b2, g2, be2, W3, b3):
    N = x.shape[0]
    E = edge_index.shape[1]
    assert N == N_NODES

    ei = edge_index.astype(jnp.int32)
    e_pad = _ceil_to(E, NW * K * 10)
    pad = e_pad - E
    padv = jnp.full((pad,), N, jnp.int32)
    src = jnp.concatenate([ei[0], padv])
    dst = jnp.concatenate([ei[1], padv])

    deg_kernel = _make_deg_kernel(e_pad)
    agg128 = _make_agg_kernel(e_pad, 128)
    agg_lat = _make_agg_kernel(e_pad, 128)

    deg_p = deg_kernel(dst)
    dis = _tc(_dis_body, jax.ShapeDtypeStruct((1, N_PAD), F32), deg_p)
    disc = dis.reshape(N_PAD, 1)

    def layer_mid(h_in_y, a_p, b, g, be, Wn):
        C_out = Wn.shape[1]
        return _tc(_comb_body, jax.ShapeDtypeStruct((N_PAD, C_out), F32),
                   a_p[0], a_p[1], h_in_y, disc, b.reshape(1, -1),
                   g.reshape(1, -1), be.reshape(1, -1), Wn)

    lat = W3.shape[1]
    W3p = jnp.zeros((W3.shape[0], 128), F32).at[:, :lat].set(W3)

    y1 = _tc(_y_body, jax.ShapeDtypeStruct((N_PAD, 128), F32), x, W1, disc)
    a1 = agg128(y1, src, dst)
    y2 = layer_mid(y1, a1, b1, g1, be1, W2)
    a2 = agg128(y2, src, dst)
    y3 = layer_mid(y2, a2, b2, g2, be2, W3p)
    a3 = agg_lat(y3, src, dst)
    out = _tc(_final_body, jax.ShapeDtypeStruct((N_NODES, lat), F32),
              a3[0], a3[1], y3, disc, b3.reshape(1, -1))
    return out


# speedup vs baseline: 1.1206x; 1.0314x over previous
import functools

import jax
import jax.numpy as jnp
from jax import lax
from jax.experimental import pallas as pl
from jax.experimental.pallas import tpu as pltpu
from jax.experimental.pallas import tpu_sc as plsc

N_NODES = 10000
N_PAD = 10240
K = 128
NC = 2
NS = 16
NW = NC * NS
ROWS_PT = N_PAD // NS
F32 = jnp.float32


def _ceil_to(a, m):
    return (a + m - 1) // m * m


def _make_deg_kernel(e_pad):
    ept = e_pad // NW
    chunks = ept // K
    seg = N_PAD // NS

    @functools.partial(
        pl.kernel,
        out_type=jax.ShapeDtypeStruct((NC, N_PAD), F32),
        mesh=plsc.VectorSubcoreMesh(core_axis_name="c", subcore_axis_name="s"),
        scratch_types=[
            pltpu.VMEM((K,), jnp.int32),
            pltpu.VMEM((N_PAD,), F32),
            pltpu.VMEM((seg,), F32),
            pltpu.VMEM((NS, seg), F32),
            pltpu.VMEM_SHARED((NS, N_PAD), F32),
        ],
        compiler_params=pltpu.CompilerParams(needs_layout_passes=False),
    )
    def deg_kernel(dst_hbm, out_hbm, idx_v, deg_local, out_local, slab_v, sh):
        cid = lax.axis_index("c")
        sid = lax.axis_index("s")
        wid = sid * NC + cid
        base = wid * ept

        zeros16 = jnp.zeros((16,), F32)
        ones16 = jnp.ones((16,), F32)

        def zbody(i, _):
            deg_local[pl.ds(i * 16, 16)] = zeros16
            return 0
        lax.fori_loop(0, N_PAD // 16, zbody, 0)

        def chunk(c, _):
            pltpu.sync_copy(dst_hbm.at[pl.ds(base + c * K, K)], idx_v)
            for j in range(K // 16):
                idx = idx_v[pl.ds(j * 16, 16)]
                plsc.addupdate_scatter(deg_local, [idx], ones16)
            return 0
        lax.fori_loop(0, chunks, chunk, 0)

        pltpu.sync_copy(deg_local, sh.at[sid])
        plsc.subcore_barrier()

        for j in range(NS):
            pltpu.sync_copy(sh.at[j, pl.ds(sid * seg, seg)], slab_v.at[j])

        def rbody(k, _):
            acc = jnp.zeros((16,), F32)
            for j in range(NS):
                acc = acc + slab_v[j, pl.ds(k * 16, 16)]
            out_local[pl.ds(k * 16, 16)] = acc
            return 0
        lax.fori_loop(0, seg // 16, rbody, 0)

        pltpu.sync_copy(out_local, out_hbm.at[cid, pl.ds(sid * seg, seg)])

    return deg_kernel


def _make_agg_kernel(e_pad, C, share0=0.875):
    KA = 64
    NB = 4
    T = e_pad // KA
    T0 = _ceil_to(int(T * share0), NS * 2 * NB)
    T1 = T - T0
    assert T1 >= 0 and T1 % (NS * 2 * NB) == 0
    cpt = (T0 // NS, T1 // NS)
    giters = (cpt[0] // (2 * NB), cpt[1] // (2 * NB))
    WB = ROWS_PT // KA

    @functools.partial(
        pl.kernel,
        out_type=jax.ShapeDtypeStruct((NC, N_PAD, C), F32),
        mesh=plsc.VectorSubcoreMesh(core_axis_name="c", subcore_axis_name="s"),
        scratch_types=[
            pltpu.VMEM((2 * NB, KA), jnp.int32),
            pltpu.VMEM((2 * NB, KA), jnp.int32),
            pltpu.VMEM((NB, KA, C), F32),
            pltpu.VMEM_SHARED((N_PAD, C), F32),
            pltpu.SemaphoreType.DMA((2 * NB,)),
            pltpu.SemaphoreType.DMA((NB,)),
            pltpu.SemaphoreType.DMA((NB,)),
        ],
    )
    def agg_kernel(y_hbm, src_hbm, dst_hbm, out_hbm, srcx_v, dstx_v, rows_v,
                   acc_sh, si, sg, ss):
        cid = lax.axis_index("c")
        sid = lax.axis_index("s")
        base_c = jnp.where(cid == 0, sid * cpt[0], T0 + sid * cpt[1])
        base_e = base_c * KA
        giter = jnp.where(cid == 0, giters[0], giters[1])
        tb = sid * ROWS_PT

        zeros16 = jnp.zeros((16,), F32)

        def zbody(i, _):
            for j in range(C // 16):
                rows_v[0, i, pl.ds(j * 16, 16)] = zeros16
            return 0
        lax.fori_loop(0, KA, zbody, 0)

        for k in range(WB):
            pltpu.sync_copy(rows_v.at[0], acc_sh.at[pl.ds(tb + k * KA, KA)])
        plsc.subcore_barrier()

        def idx_cp(c, b):
            return [pltpu.make_async_copy(
                src_hbm.at[pl.ds(base_e + c * KA, KA)], srcx_v.at[b],
                si.at[b]), pltpu.make_async_copy(
                dst_hbm.at[pl.ds(base_e + c * KA, KA)], dstx_v.at[b],
                si.at[b])]

        def gat_cp(b, r):
            return pltpu.make_async_copy(
                y_hbm.at[srcx_v.at[b]], rows_v.at[r], sg.at[r])

        def sca_cp(b, r):
            return pltpu.make_async_copy(
                rows_v.at[r], acc_sh.at[dstx_v.at[b]], ss.at[r])

        def body(g, _):
            base = g * 2 * NB
            @pl.when(g > 0)
            def _():
                for b in range(NB):
                    sca_cp(NB + b, b).wait()
            for b in range(2 * NB):
                for cp in idx_cp(base + b, b):
                    cp.start()
            for b in range(NB):
                for cp in idx_cp(base + b, b):
                    cp.wait()
                gat_cp(b, b).start()
            for b in range(NB):
                gat_cp(b, b).wait()
                sca_cp(b, b).start(add=True)
            for b in range(NB):
                sca_cp(b, b).wait()
                for cp in idx_cp(base + NB + b, NB + b):
                    cp.wait()
                gat_cp(NB + b, b).start()
            for b in range(NB):
                gat_cp(NB + b, b).wait()
                sca_cp(NB + b, b).start(add=True)
            return 0
        lax.fori_loop(0, giter, body, 0)
        for b in range(NB):
            sca_cp(NB + b, b).wait()

        plsc.subcore_barrier()

        def st_cp(k, r):
            return pltpu.make_async_copy(
                rows_v.at[r], out_hbm.at[cid, pl.ds(tb + k * KA, KA)],
                sg.at[r])

        for k in range(WB):
            r = k % 2
            if k >= 2:
                st_cp(k - 2, r).wait()
            pltpu.sync_copy(acc_sh.at[pl.ds(tb + k * KA, KA)], rows_v.at[r])
            st_cp(k, r).start()
        for k in (WB - 2, WB - 1):
            st_cp(k, k % 2).wait()

    return agg_kernel


def _dis_body(dp_ref, dis_ref):
    dp = dp_ref[...]
    deg = dp[0:1, :] + dp[1:2, :] + 1.0
    dis_ref[...] = lax.rsqrt(deg)


def _y_body(x_ref, w_ref, d_ref, y_ref):
    xw = jnp.dot(x_ref[...], w_ref[...], preferred_element_type=F32)
    y_ref[0:N_NODES, :] = xw * d_ref[0:N_NODES, :]
    y_ref[N_NODES:N_PAD, :] = jnp.zeros((N_PAD - N_NODES, y_ref.shape[1]), F32)


def _comb_body(a0_ref, a1_ref, y_ref, d_ref, b_ref, g_ref, be_ref, w_ref,
               o_ref):
    d = d_ref[0:N_NODES, :]
    t = (a0_ref[0:N_NODES, :] + a1_ref[0:N_NODES, :] + y_ref[0:N_NODES, :]) \
        * d + b_ref[...]
    mean = jnp.mean(t, axis=0, keepdims=True)
    tc = t - mean
    var = jnp.mean(tc * tc, axis=0, keepdims=True)
    h = g_ref[...] * tc * lax.rsqrt(var + 1e-5) + be_ref[...]
    h = jnp.maximum(h, 0.0)
    o_ref[0:N_NODES, :] = jnp.dot(h, w_ref[...], preferred_element_type=F32) \
        * d
    o_ref[N_NODES:N_PAD, :] = jnp.zeros((N_PAD - N_NODES, o_ref.shape[1]), F32)


def _final_body(a0_ref, a1_ref, y_ref, d_ref, b_ref, o_ref):
    C = o_ref.shape[1]
    o_ref[...] = (a0_ref[0:N_NODES, 0:C] + a1_ref[0:N_NODES, 0:C]
                  + y_ref[0:N_NODES, 0:C]) * d_ref[0:N_NODES, :] + b_ref[...]


def _tc(body, out_shape, *args):
    return pl.pallas_call(body, out_shape=out_shape)(*args)


def kernel(x, edge_index, W1, b1, g1, be1, W2, b2, g2, be2, W3, b3):
    N = x.shape[0]
    E = edge_index.shape[1]
    assert N == N_NODES

    ei = edge_index.astype(jnp.int32)
    e_pad = _ceil_to(E, NW * K * 10)
    pad = e_pad - E
    padv = jnp.full((pad,), N, jnp.int32)
    src = jnp.concatenate([ei[0], padv])
    dst = jnp.concatenate([ei[1], padv])

    deg_kernel = _make_deg_kernel(e_pad)
    agg128 = _make_agg_kernel(e_pad, 128)
    agg_lat = _make_agg_kernel(e_pad, 128)

    deg_p = deg_kernel(dst)
    dis = _tc(_dis_body, jax.ShapeDtypeStruct((1, N_PAD), F32), deg_p)
    disc = dis.reshape(N_PAD, 1)

    def layer_mid(h_in_y, a_p, b, g, be, Wn):
        C_out = Wn.shape[1]
        return _tc(_comb_body, jax.ShapeDtypeStruct((N_PAD, C_out), F32),
                   a_p[0], a_p[1], h_in_y, disc, b.reshape(1, -1),
                   g.reshape(1, -1), be.reshape(1, -1), Wn)

    lat = W3.shape[1]
    W3p = jnp.zeros((W3.shape[0], 128), F32).at[:, :lat].set(W3)

    y1 = _tc(_y_body, jax.ShapeDtypeStruct((N_PAD, 128), F32), x, W1, disc)
    a1 = agg128(y1, src, dst)
    y2 = layer_mid(y1, a1, b1, g1, be1, W2)
    a2 = agg128(y2, src, dst)
    y3 = layer_mid(y2, a2, b2, g2, be2, W3p)
    a3 = agg_lat(y3, src, dst)
    out = _tc(_final_body, jax.ShapeDtypeStruct((N_NODES, lat), F32),
              a3[0], a3[1], y3, disc, b3.reshape(1, -1))
    return out
